# baseline (speedup 1.0000x reference)
import functools

import jax
import jax.numpy as jnp
from jax import lax
from jax.experimental import pallas as pl
from jax.experimental.pallas import tpu as pltpu
from jax.experimental.pallas import tpu_sc as plsc

_ETA = 0.8
_GAMMA = 0.9

_NC = 2
_NS = 16
_NW = _NC * _NS
_L = 16
_CHUNK = 4096
_NBUF = 4
_SPN = _CHUNK // 2048
_UG = _CHUNK // 128
_PARTS = 16384 // _CHUNK


@functools.lru_cache(maxsize=None)
def _build(n):
    per_w = n // _NW
    nchunks = per_w // _CHUNK
    assert per_w % _CHUNK == 0
    ng = n // 128

    mesh = plsc.VectorSubcoreMesh(
        core_axis_name="c", subcore_axis_name="s",
        num_cores=_NC, num_subcores=_NS)

    scratch = []
    for _ in range(_NBUF):
        scratch += [
            pltpu.VMEM((_UG, 2, 128), jnp.float32),
            pltpu.VMEM((_UG, 2, 128), jnp.float32),
            pltpu.VMEM((16, _SPN, 128), jnp.int32),
            pltpu.VMEM((16, _SPN, 128), jnp.int32),
            pltpu.VMEM((16, _SPN, 128), jnp.float32),
            pltpu.SemaphoreType.DMA,
            pltpu.SemaphoreType.DMA,
        ]

    @functools.partial(
        pl.kernel,
        out_type=jax.ShapeDtypeStruct((2, ng, 2, 128), jnp.float32),
        mesh=mesh,
        scratch_types=scratch,
        compiler_params=pltpu.CompilerParams(needs_layout_passes=False),
    )
    def run(q_hbm, a_hbm, b_hbm, p_hbm, out_hbm, *bufs):
        wid = lax.axis_index("s") * _NC + lax.axis_index("c")
        slots = [bufs[i * 7:(i + 1) * 7] for i in range(_NBUF)]

        def chunk_coords(g):
            cid = wid * nchunks + g
            band = cid // _PARTS
            part = cid % _PARTS
            u0 = cid * _UG
            return band, part, u0

        def start_in(g, slot):
            q0b, q1b, ab, bb, pb, insem, _ = slot
            band, part, u0 = chunk_coords(g)
            s0 = _SPN * part
            return [
                pltpu.async_copy(q_hbm.at[0, pl.ds(u0, _UG)], q0b, insem),
                pltpu.async_copy(q_hbm.at[1, pl.ds(u0, _UG)], q1b, insem),
                pltpu.async_copy(a_hbm.at[band, :, pl.ds(s0, _SPN), :], ab, insem),
                pltpu.async_copy(b_hbm.at[band, :, pl.ds(s0, _SPN), :], bb, insem),
                pltpu.async_copy(p_hbm.at[band, :, pl.ds(s0, _SPN), :], pb, insem),
            ]

        def start_out(g, slot):
            q0b, q1b = slot[0], slot[1]
            outsem = slot[6]
            _, _, u0 = chunk_coords(g)
            return [
                pltpu.async_copy(q0b, out_hbm.at[0, pl.ds(u0, _UG)], outsem),
                pltpu.async_copy(q1b, out_hbm.at[1, pl.ds(u0, _UG)], outsem),
            ]

        def compute(slot):
            q0b, q1b, ab, bb, pb, _, _ = slot
            lane = lax.iota(jnp.int32, _L)

            @plsc.parallel_loop(0, _CHUNK // _L, unroll=8)
            def body(j):
                cc = j // (8 * _SPN)
                sp = (j // 8) % _SPN
                l0 = (j % 8) * _L
                u = sp * 16 + cc
                a = ab[cc, sp, pl.ds(l0, _L)]
                b = bb[cc, sp, pl.ds(l0, _L)]
                p = pb[cc, sp, pl.ds(l0, _L)]
                q00 = q0b[u, 0, pl.ds(l0, _L)]
                q01 = q0b[u, 1, pl.ds(l0, _L)]
                q10 = q1b[u, 0, pl.ds(l0, _L)]
                q11 = q1b[u, 1, pl.ds(l0, _L)]
                b0 = b == 0
                a0 = a == 0
                maxv = jnp.where(b0, jnp.maximum(q00, q01),
                                 jnp.maximum(q10, q11))
                qsel = jnp.where(a0, jnp.where(b0, q00, q01),
                                 jnp.where(b0, q10, q11))
                upd = (1.0 - _ETA) * qsel + _ETA * (p + _GAMMA * maxv)
                uv = jnp.broadcast_to(u, (_L,))
                lv = lane + l0
                plsc.store_scatter(q0b, [uv, b, lv], upd, mask=a0)
                plsc.store_scatter(q1b, [uv, b, lv], upd, mask=~a0)

        in_descs = [None] * _NBUF
        out_descs = [None] * _NBUF
        prime = min(_NBUF - 1, nchunks)
        for g in range(prime):
            in_descs[g] = start_in(g, slots[g])
        for g in range(nchunks):
            s = g % _NBUF
            np_ = g + prime
            ns = np_ % _NBUF
            if np_ < nchunks:
                if out_descs[ns] is not None:
                    for d in out_descs[ns]:
                        d.wait()
                    out_descs[ns] = None
                in_descs[ns] = start_in(np_, slots[ns])
            for d in in_descs[s]:
                d.wait()
            compute(slots[s])
            out_descs[s] = start_out(g, slots[s])
        for ods in out_descs:
            if ods is not None:
                for d in ods:
                    d.wait()

    return run


def kernel(type_t_matrix, type_t1_matrix, Q_tensor, profit_matrix):
    n = Q_tensor.shape[0]
    ng = n // 128
    qv = Q_tensor.reshape(ng, 128, 2, 2).transpose(2, 0, 3, 1)
    av = type_t_matrix.reshape(256, 8, 16, 128).transpose(0, 2, 1, 3)
    bv = type_t1_matrix.reshape(256, 8, 16, 128).transpose(0, 2, 1, 3)
    pv = profit_matrix.reshape(256, 8, 16, 128).transpose(0, 2, 1, 3)
    out = _build(n)(qv, av.astype(jnp.int32), bv.astype(jnp.int32), pv)
    return out.transpose(1, 3, 0, 2).reshape(n, 2, 2)


# speedup vs baseline: 1.0144x; 1.0144x over previous
import functools

import jax
import jax.numpy as jnp
from jax import lax
from jax.experimental import pallas as pl
from jax.experimental.pallas import tpu as pltpu
from jax.experimental.pallas import tpu_sc as plsc

_ETA = 0.8
_GAMMA = 0.9

_NC = 2
_NS = 16
_NW = _NC * _NS
_L = 16
_CHUNK = 8192
_HALF = 4096
_HUG = _HALF // 128
_NQ = 4
_NABP = 2


@functools.lru_cache(maxsize=None)
def _build(n):
    per_w = n // _NW
    nchunks = per_w // _CHUNK
    nhalves = 2 * nchunks
    assert per_w % _CHUNK == 0
    ng = n // 128

    mesh = plsc.VectorSubcoreMesh(
        core_axis_name="c", subcore_axis_name="s",
        num_cores=_NC, num_subcores=_NS)

    scratch = []
    for _ in range(_NQ):
        scratch += [
            pltpu.VMEM((_HUG, 2, 128), jnp.float32),
            pltpu.VMEM((_HUG, 2, 128), jnp.float32),
            pltpu.SemaphoreType.DMA,
            pltpu.SemaphoreType.DMA,
        ]
    for _ in range(_NABP):
        scratch += [
            pltpu.VMEM((16, 4, 128), jnp.int32),
            pltpu.VMEM((16, 4, 128), jnp.int32),
            pltpu.VMEM((16, 4, 128), jnp.float32),
            pltpu.SemaphoreType.DMA,
        ]

    @functools.partial(
        pl.kernel,
        out_type=jax.ShapeDtypeStruct((2, ng, 2, 128), jnp.float32),
        mesh=mesh,
        scratch_types=scratch,
        compiler_params=pltpu.CompilerParams(needs_layout_passes=False),
    )
    def run(q_hbm, a_hbm, b_hbm, p_hbm, out_hbm, *bufs):
        wid = lax.axis_index("s") * _NC + lax.axis_index("c")
        qslots = [bufs[i * 4:(i + 1) * 4] for i in range(_NQ)]
        abase = _NQ * 4
        abpslots = [bufs[abase + i * 4:abase + (i + 1) * 4]
                    for i in range(_NABP)]

        def start_in_q(hid, slot):
            q0b, q1b, insem, _ = slot
            u0 = (wid * nhalves + hid) * _HUG
            return [
                pltpu.async_copy(q_hbm.at[0, pl.ds(u0, _HUG)], q0b, insem),
                pltpu.async_copy(q_hbm.at[1, pl.ds(u0, _HUG)], q1b, insem),
            ]

        def start_out(hid, slot):
            q0b, q1b, _, outsem = slot
            u0 = (wid * nhalves + hid) * _HUG
            return [
                pltpu.async_copy(q0b, out_hbm.at[0, pl.ds(u0, _HUG)], outsem),
                pltpu.async_copy(q1b, out_hbm.at[1, pl.ds(u0, _HUG)], outsem),
            ]

        def start_in_abp(g, slot):
            ab, bb, pb, insem = slot
            cid = wid * nchunks + g
            band = cid // 2
            s0 = 4 * (cid % 2)
            return [
                pltpu.async_copy(a_hbm.at[band, :, pl.ds(s0, 4), :], ab, insem),
                pltpu.async_copy(b_hbm.at[band, :, pl.ds(s0, 4), :], bb, insem),
                pltpu.async_copy(p_hbm.at[band, :, pl.ds(s0, 4), :], pb, insem),
            ]

        def compute(qslot, abpslot, h):
            q0b, q1b = qslot[0], qslot[1]
            ab, bb, pb = abpslot[0], abpslot[1], abpslot[2]
            lane = lax.iota(jnp.int32, _L)

            @plsc.parallel_loop(0, _HALF // _L, unroll=8)
            def body(j):
                cc = j // 16
                sp2 = (j // 8) % 2
                sp = 2 * h + sp2
                l0 = (j % 8) * _L
                u = sp2 * 16 + cc
                a = ab[cc, sp, pl.ds(l0, _L)]
                b = bb[cc, sp, pl.ds(l0, _L)]
                p = pb[cc, sp, pl.ds(l0, _L)]
                q00 = q0b[u, 0, pl.ds(l0, _L)]
                q01 = q0b[u, 1, pl.ds(l0, _L)]
                q10 = q1b[u, 0, pl.ds(l0, _L)]
                q11 = q1b[u, 1, pl.ds(l0, _L)]
                b0 = b == 0
                a0 = a == 0
                maxv = jnp.where(b0, jnp.maximum(q00, q01),
                                 jnp.maximum(q10, q11))
                qsel = jnp.where(a0, jnp.where(b0, q00, q01),
                                 jnp.where(b0, q10, q11))
                upd = (1.0 - _ETA) * qsel + _ETA * (p + _GAMMA * maxv)
                uv = jnp.broadcast_to(u, (_L,))
                lv = lane + l0
                plsc.store_scatter(q0b, [uv, b, lv], upd, mask=a0)
                plsc.store_scatter(q1b, [uv, b, lv], upd, mask=~a0)

        in_q = [None] * _NQ
        out_q = [None] * _NQ
        in_abp = [None] * _NABP

        in_abp[0] = start_in_abp(0, abpslots[0])
        prime = min(_NQ - 1, nhalves)
        for hid in range(prime):
            in_q[hid] = start_in_q(hid, qslots[hid])

        for hid in range(nhalves):
            g, h = hid // 2, hid % 2
            qs = hid % _NQ
            if h == 0 and g + 1 < nchunks:
                in_abp[(g + 1) % _NABP] = start_in_abp(
                    g + 1, abpslots[(g + 1) % _NABP])
            nh = hid + prime
            if nh < nhalves:
                ns = nh % _NQ
                if out_q[ns] is not None:
                    for d in out_q[ns]:
                        d.wait()
                    out_q[ns] = None
                in_q[ns] = start_in_q(nh, qslots[ns])
            for d in in_q[qs]:
                d.wait()
            if h == 0:
                for d in in_abp[g % _NABP]:
                    d.wait()
                in_abp[g % _NABP] = []
            compute(qslots[qs], abpslots[g % _NABP], h)
            out_q[qs] = start_out(hid, qslots[qs])

        for ods in out_q:
            if ods is not None:
                for d in ods:
                    d.wait()

    return run


def kernel(type_t_matrix, type_t1_matrix, Q_tensor, profit_matrix):
    n = Q_tensor.shape[0]
    ng = n // 128
    qv = Q_tensor.reshape(ng, 128, 2, 2).transpose(2, 0, 3, 1)
    av = type_t_matrix.reshape(256, 8, 16, 128).transpose(0, 2, 1, 3)
    bv = type_t1_matrix.reshape(256, 8, 16, 128).transpose(0, 2, 1, 3)
    pv = profit_matrix.reshape(256, 8, 16, 128).transpose(0, 2, 1, 3)
    out = _build(n)(qv, av.astype(jnp.int32), bv.astype(jnp.int32), pv)
    return out.transpose(1, 3, 0, 2).reshape(n, 2, 2)
